# Initial kernel scaffold; baseline (speedup 1.0000x reference)
import jax
import jax.numpy as jnp
from jax.experimental import pallas as pl


def kernel(x, edge_index, edge_values, W):
    raise NotImplementedError("write your pallas kernel here")


# speedup vs baseline: 5.2000x; 5.2000x over previous
import functools

import jax
import jax.numpy as jnp
from jax import lax
from jax.experimental import pallas as pl
from jax.experimental.pallas import tpu as pltpu
from jax.experimental.pallas import tpu_sc as plsc

NC = 2
NS = 16
NW = NC * NS
L = 16
CHUNK = 128


def _matmul_body(x_ref, w_ref, h_ref):
    h_ref[...] = lax.dot_general(
        x_ref[...], w_ref[...],
        dimension_numbers=(((1,), (1,)), ((), ())),
        preferred_element_type=jnp.float32,
    )


def _make_sc_sum(NPAD, D):
    rpt = NPAD // NW
    CH = 80
    mesh = plsc.VectorSubcoreMesh(core_axis_name="c", subcore_axis_name="s",
                                  num_cores=NC, num_subcores=NS)

    @functools.partial(
        pl.kernel,
        out_type=jax.ShapeDtypeStruct((NPAD, D), jnp.float32),
        mesh=mesh,
        scratch_types=[
            pltpu.VMEM((CH, D), jnp.float32),
            pltpu.VMEM((CH, D), jnp.float32),
        ],
    )
    def sc_sum(p_hbm, out_hbm, b0, b1):
        c = lax.axis_index("c")
        s = lax.axis_index("s")
        wid = s * NC + c
        for j in range(rpt // CH):
            r0 = wid * rpt + j * CH
            pltpu.sync_copy(p_hbm.at[0, pl.ds(r0, CH)], b0)
            pltpu.sync_copy(p_hbm.at[1, pl.ds(r0, CH)], b1)

            def body(i, carry):
                for kk in range(D // L):
                    sl = pl.ds(kk * L, L)
                    b0[i, sl] = b0[i, sl] + b1[i, sl]
                return carry

            lax.fori_loop(0, CH, body, 0)
            pltpu.sync_copy(b0, out_hbm.at[pl.ds(r0, CH)])

    return sc_sum


def _make_agg(NPAD, D, E):
    num_chunks = E // CHUNK
    rows_per_tile = NPAD // NS
    iters = (num_chunks + NW - 1) // NW
    mesh = plsc.VectorSubcoreMesh(core_axis_name="c", subcore_axis_name="s",
                                  num_cores=NC, num_subcores=NS)

    @functools.partial(
        pl.kernel,
        out_type=jax.ShapeDtypeStruct((NC, NPAD, D), jnp.float32),
        mesh=mesh,
        scratch_types=[
            pltpu.VMEM((CHUNK,), jnp.int32),
            pltpu.VMEM((CHUNK,), jnp.int32),
            pltpu.VMEM((CHUNK,), jnp.float32),
            pltpu.VMEM((CHUNK, D), jnp.float32),
            pltpu.VMEM_SHARED((NPAD, D), jnp.float32),
            pltpu.SemaphoreType.DMA,
        ],
    )
    def agg(h_hbm, col_hbm, row_hbm, val_hbm, out_hbm,
            colv, rowv, valv, rows, acc, sem):
        c = lax.axis_index("c")
        s = lax.axis_index("s")
        wid = s * NC + c

        zero = jnp.zeros((L,), jnp.float32)

        def zbody(i, carry):
            for k in range(D // L):
                rows[i, pl.ds(k * L, L)] = zero
            return carry

        lax.fori_loop(0, CHUNK, zbody, 0)
        for j in range(rows_per_tile // CHUNK):
            pltpu.sync_copy(rows.at[:],
                            acc.at[pl.ds(s * rows_per_tile + j * CHUNK, CHUNK)])
        plsc.subcore_barrier()

        def chunk_body(t, carry):
            ci = wid + t * NW

            @pl.when(ci < num_chunks)
            def _():
                base = ci * CHUNK
                pltpu.sync_copy(col_hbm.at[pl.ds(base, CHUNK)], colv)
                pltpu.sync_copy(row_hbm.at[pl.ds(base, CHUNK)], rowv)
                pltpu.sync_copy(val_hbm.at[pl.ds(base, CHUNK)], valv)
                pltpu.async_copy(h_hbm.at[colv], rows, sem).wait()

                def sbody(g, carry2):
                    vv = valv[pl.ds(g * L, L)]
                    for j in range(L):
                        e = g * L + j
                        v = vv[j]
                        for k in range(D // L):
                            sl = pl.ds(k * L, L)
                            rows[e, sl] = rows[e, sl] * v
                    return carry2

                lax.fori_loop(0, CHUNK // L, sbody, 0)
                pltpu.sync_copy(rows, acc.at[rowv], add=True)

            return carry

        lax.fori_loop(0, iters, chunk_body, 0)
        plsc.subcore_barrier()

        pltpu.sync_copy(acc.at[pl.ds(s * rows_per_tile, rows_per_tile)],
                        out_hbm.at[c, pl.ds(s * rows_per_tile, rows_per_tile)])

    return agg


def kernel(x, edge_index, edge_values, W):
    N, D = x.shape
    E = edge_values.shape[0]
    row = edge_index[0]
    col = edge_index[1]
    NPAD = ((N + 2559) // 2560) * 2560

    rb = 1000
    h = pl.pallas_call(
        _matmul_body,
        grid=(N // rb,),
        in_specs=[
            pl.BlockSpec((rb, D), lambda i: (i, 0)),
            pl.BlockSpec((D, D), lambda i: (0, 0)),
        ],
        out_specs=pl.BlockSpec((rb, D), lambda i: (i, 0)),
        out_shape=jax.ShapeDtypeStruct((N, D), jnp.float32),
    )(x, W)

    partials = _make_agg(NPAD, D, E)(h, col, row, edge_values)
    out = _make_sc_sum(NPAD, D)(partials)
    return out[:N]
